# baseline (speedup 1.0000x reference)
import functools

import jax
import jax.numpy as jnp
from jax import lax
from jax.experimental import pallas as pl
from jax.experimental.pallas import tpu as pltpu
from jax.experimental.pallas import tpu_sc as plsc

EPS = 1e-5


def _ln_table_body(table_ref, w_ref, b_ref, out_ref):
    t = table_ref[...]
    mean = jnp.mean(t, axis=-1, keepdims=True)
    var = jnp.mean(jnp.square(t - mean), axis=-1, keepdims=True)
    out_ref[...] = (t - mean) * lax.rsqrt(var + EPS) * w_ref[...] + b_ref[...]


def _normalize_table(table, ln_weight, ln_bias):
    rows, hidden = table.shape
    return pl.pallas_call(
        _ln_table_body,
        out_shape=jax.ShapeDtypeStruct((rows, hidden), jnp.float32),
    )(table, ln_weight.reshape(1, hidden), ln_bias.reshape(1, hidden))


def _make_sc_gather(tokens, hidden, chunk):
    info = plsc.get_sparse_core_info()
    nc, ns = info.num_cores, info.num_subcores
    nw = nc * ns
    per_w = tokens // nw
    nchunks = per_w // chunk
    mesh = plsc.VectorSubcoreMesh(core_axis_name="c", subcore_axis_name="s")

    @functools.partial(
        pl.kernel,
        mesh=mesh,
        out_type=jax.ShapeDtypeStruct((tokens, hidden), jnp.float32),
        scratch_types=[
            pltpu.VMEM((per_w,), jnp.int32),
            pltpu.VMEM((chunk, hidden), jnp.float32),
            pltpu.SemaphoreType.DMA,
        ],
    )
    def sc_gather(normed_hbm, idx_hbm, out_hbm, idx_v, rows_v, sem):
        wid = lax.axis_index("s") * nc + lax.axis_index("c")
        base = wid * per_w
        pltpu.sync_copy(idx_hbm.at[pl.ds(base, per_w)], idx_v)

        def body(g, carry):
            start = g * chunk
            pltpu.async_copy(
                normed_hbm.at[idx_v.at[pl.ds(start, chunk)]], rows_v, sem
            ).wait()
            pltpu.sync_copy(rows_v, out_hbm.at[pl.ds(base + start, chunk)])
            return carry

        lax.fori_loop(0, nchunks, body, 0)

    return sc_gather


def kernel(type_token, table, ln_weight, ln_bias):
    b, s = type_token.shape
    rows, hidden = table.shape
    tokens = b * s
    normed = _normalize_table(table, ln_weight, ln_bias)
    idx = type_token.reshape(tokens).astype(jnp.int32)
    out = _make_sc_gather(tokens, hidden, chunk=32)(normed, idx)
    return out.reshape(b, s, hidden)


# speedup vs baseline: 6.3679x; 6.3679x over previous
import functools

import jax
import jax.numpy as jnp
from jax import lax
from jax.experimental import pallas as pl
from jax.experimental.pallas import tpu as pltpu
from jax.experimental.pallas import tpu_sc as plsc

EPS = 1e-5


def _ln_table_body(table_ref, w_ref, b_ref, out_ref):
    t = table_ref[...]
    mean = jnp.mean(t, axis=-1, keepdims=True)
    var = jnp.mean(jnp.square(t - mean), axis=-1, keepdims=True)
    out_ref[...] = (t - mean) * lax.rsqrt(var + EPS) * w_ref[...] + b_ref[...]


def _normalize_table(table, ln_weight, ln_bias):
    rows, hidden = table.shape
    return pl.pallas_call(
        _ln_table_body,
        out_shape=jax.ShapeDtypeStruct((rows, hidden), jnp.float32),
    )(table, ln_weight.reshape(1, hidden), ln_bias.reshape(1, hidden))


def _make_sc_gather(tokens, hidden, rows):
    info = plsc.get_sparse_core_info()
    nc, ns, nl = info.num_cores, info.num_subcores, info.num_lanes
    nw = nc * ns
    per_w = tokens // nw
    ngroups = per_w // nl
    mesh = plsc.VectorSubcoreMesh(core_axis_name="c", subcore_axis_name="s")

    @functools.partial(
        pl.kernel,
        mesh=mesh,
        out_type=jax.ShapeDtypeStruct((tokens, hidden), jnp.float32),
        scratch_types=[
            pltpu.VMEM((rows, hidden), jnp.float32),
            pltpu.VMEM((per_w,), jnp.int32),
            pltpu.VMEM((nl, hidden), jnp.float32),
            pltpu.SemaphoreType.DMA,
        ],
    )
    def sc_gather(normed_hbm, idx_hbm, out_hbm, tab_v, idx_v, drain_v, osem):
        wid = lax.axis_index("s") * nc + lax.axis_index("c")
        base = wid * per_w
        pltpu.sync_copy(normed_hbm, tab_v)
        pltpu.sync_copy(idx_hbm.at[pl.ds(base, per_w)], idx_v)

        def group(g, carry):
            iv = idx_v[pl.ds(g * nl, nl)]
            for j in range(nl):
                t = g * nl + j
                rid = iv[j]
                pltpu.async_copy(
                    tab_v.at[pl.ds(rid, 1)],
                    out_hbm.at[pl.ds(base + t, 1)],
                    osem,
                )
            @pl.when(g > 0)
            def _():
                pltpu.make_async_copy(
                    out_hbm.at[pl.ds(base, nl)], drain_v, osem
                ).wait()
            return carry

        lax.fori_loop(0, ngroups, group, 0)
        pltpu.make_async_copy(out_hbm.at[pl.ds(base, nl)], drain_v, osem).wait()

    return sc_gather


def kernel(type_token, table, ln_weight, ln_bias):
    b, s = type_token.shape
    rows, hidden = table.shape
    tokens = b * s
    normed = _normalize_table(table, ln_weight, ln_bias)
    idx = type_token.reshape(tokens).astype(jnp.int32)
    out = _make_sc_gather(tokens, hidden, rows)(normed, idx)
    return out.reshape(b, s, hidden)
